# baseline (speedup 1.0000x reference)
import jax
import jax.numpy as jnp
from jax.experimental import pallas as pl

_D = 768
_K = 1024
_CHUNK = 512


def _vq_block(x_ref, w_ref, b_ref, cb_ref, out_ref):
    cb = cb_ref[...]
    cbn = jnp.sum(cb * cb, axis=1, keepdims=True)
    w = w_ref[...]
    t_len = x_ref.shape[2]
    for h in range(t_len // _CHUNK):
        x = x_ref[0, :, h * _CHUNK:(h + 1) * _CHUNK]
        y = jnp.dot(w, x, preferred_element_type=jnp.float32) + b_ref[...]
        s = jnp.dot(cb, y, preferred_element_type=jnp.float32)
        xn = jnp.sum(y * y, axis=0, keepdims=True)
        dist = (xn - 2.0 * s) + cbn
        idx = jnp.argmin(dist, axis=0)[None, :].astype(jnp.int32)
        out_ref[:, h * _CHUNK:(h + 1) * _CHUNK] = idx


def kernel(ssl_content, proj_w, proj_b, codebook):
    t_len = ssl_content.shape[2]
    b2 = proj_b[:, None]
    tile = 2048
    return pl.pallas_call(
        _vq_block,
        grid=(t_len // tile,),
        in_specs=[
            pl.BlockSpec((1, _D, tile), lambda i: (0, 0, i)),
            pl.BlockSpec((_D, _D), lambda i: (0, 0)),
            pl.BlockSpec((_D, 1), lambda i: (0, 0)),
            pl.BlockSpec((_K, _D), lambda i: (0, 0)),
        ],
        out_specs=pl.BlockSpec((1, tile), lambda i: (0, i)),
        out_shape=jax.ShapeDtypeStruct((1, t_len), jnp.int32),
    )(ssl_content, proj_w, b2, codebook)


# speedup vs baseline: 1.0494x; 1.0494x over previous
import jax
import jax.numpy as jnp
from jax.experimental import pallas as pl

_D = 768
_K = 1024
_CHUNK = 1024


def _vq_block(x_ref, w_ref, b_ref, cb_ref, out_ref):
    cb = cb_ref[...]
    cbn = jnp.sum(cb * cb, axis=1, keepdims=True)
    w = w_ref[...]
    t_len = x_ref.shape[2]
    n_ch = t_len // _CHUNK
    ys = []
    for h in range(n_ch):
        x = x_ref[0, :, h * _CHUNK:(h + 1) * _CHUNK]
        ys.append(jnp.dot(w, x, preferred_element_type=jnp.float32) + b_ref[...])
    ss = [jnp.dot(cb, y, preferred_element_type=jnp.float32) for y in ys]
    for h in range(n_ch):
        xn = jnp.sum(ys[h] * ys[h], axis=0, keepdims=True)
        dist = (xn - 2.0 * ss[h]) + cbn
        idx = jnp.argmin(dist, axis=0)[None, :].astype(jnp.int32)
        out_ref[:, h * _CHUNK:(h + 1) * _CHUNK] = idx


def kernel(ssl_content, proj_w, proj_b, codebook):
    t_len = ssl_content.shape[2]
    b2 = proj_b[:, None]
    tile = 2048
    return pl.pallas_call(
        _vq_block,
        grid=(t_len // tile,),
        in_specs=[
            pl.BlockSpec((1, _D, tile), lambda i: (0, 0, i)),
            pl.BlockSpec((_D, _D), lambda i: (0, 0)),
            pl.BlockSpec((_D, 1), lambda i: (0, 0)),
            pl.BlockSpec((_K, _D), lambda i: (0, 0)),
        ],
        out_specs=pl.BlockSpec((1, tile), lambda i: (0, i)),
        out_shape=jax.ShapeDtypeStruct((1, t_len), jnp.int32),
    )(ssl_content, proj_w, b2, codebook)
